# baseline (speedup 1.0000x reference)
import jax
import jax.numpy as jnp
from jax.experimental import pallas as pl

_TEST = 0.0
_N = 131072
_D = 64
_BR = 2048


def _body(c_ref, d_ref, out_ref):
    cv = c_ref[...]
    dv = d_ref[...]
    tc = cv[:, 0:1]
    td = dv[:, 0:1]

    lo = tc - td
    hi = tc + td
    left_mask = lo < _TEST
    right_mask = hi >= _TEST

    lc = (lo + jnp.minimum(hi, _TEST)) / 2.0
    ld = (jnp.minimum(lc + td, _TEST) - (lc - td)) / 2.0
    rc = (jnp.maximum(lo, _TEST) + hi) / 2.0
    rd = (rc + td - jnp.maximum(rc - td, _TEST)) / 2.0

    both = left_mask & right_mask
    j_lo = jnp.minimum(lc - ld, rc - rd)
    j_hi = jnp.maximum(lc + ld, rc + rd)
    jc = (j_lo + j_hi) / 2.0
    jd = (j_hi - j_lo) / 2.0

    new_tc = jnp.where(both, jc, jnp.where(left_mask, lc, jnp.where(right_mask, rc, tc)))
    new_td = jnp.where(both, jd, jnp.where(left_mask, ld, jnp.where(right_mask, rd, td)))

    col0 = jax.lax.broadcasted_iota(jnp.int32, (1, _D), 1) == 0
    out_ref[0] = jnp.where(col0, new_tc, cv)
    out_ref[1] = jnp.where(col0, new_td, dv)


def kernel(c, delta):
    return pl.pallas_call(
        _body,
        grid=(_N // _BR,),
        in_specs=[
            pl.BlockSpec((_BR, _D), lambda i: (i, 0)),
            pl.BlockSpec((_BR, _D), lambda i: (i, 0)),
        ],
        out_specs=pl.BlockSpec((2, _BR, _D), lambda i: (0, i, 0)),
        out_shape=jax.ShapeDtypeStruct((2, _N, _D), jnp.float32),
    )(c, delta)
